# baseline (speedup 1.0000x reference)
import functools

import jax
import jax.numpy as jnp
from jax import lax
from jax.experimental import pallas as pl
from jax.experimental.pallas import tpu as pltpu
from jax.experimental.pallas import tpu_sc as plsc

_NC = 2
_NS = 16
_L = 16
_NW = _NC * _NS
_QPW = 2
_H = 512
_W = 512
_HW = _H * _W
_CH = 32768
_NCH = _HW // _CH
_VIT = _CH // _L
_LN2 = 0.6931471805599453


def _sc_body(scores_hbm, off_hbm, scl_hbm, out_hbm,
             buf0, buf1, row0, row1, row2, resbuf, sem0, sem1, semr):
    wid = lax.axis_index("s") * _NC + lax.axis_index("c")
    lane = lax.iota(jnp.int32, _L)
    bufs = (buf0, buf1)
    sems = (sem0, sem1)

    tasks = [(q, c) for q in range(_QPW) for c in range(_NCH)]

    def start(t):
        q, c = tasks[t]
        src = scores_hbm.at[wid * _QPW + q, pl.ds(c * _CH, _CH)]
        return pltpu.async_copy(src, bufs[t % 2], sems[t % 2])

    cps = [None] * len(tasks)
    cps[0] = start(0)
    cps[1] = start(1)

    neg_inf = jnp.full((_L,), -jnp.inf, jnp.float32)
    zero_i = jnp.zeros((_L,), jnp.int32)
    cmax, cidx = neg_inf, zero_i
    res = jnp.zeros((_L,), jnp.float32)

    for t, (q, c) in enumerate(tasks):
        cps[t].wait()
        buf = bufs[t % 2]
        cbase = c * _VIT

        def inner(i, carry, buf=buf, cbase=cbase):
            m, ix = carry
            v = buf[pl.ds(i * _L, _L)]
            g = v > m
            m = jnp.where(g, v, m)
            ix = jnp.where(g, cbase + i, ix)
            return m, ix

        cmax, cidx = lax.fori_loop(0, _VIT, inner, (cmax, cidx))
        if t + 2 < len(tasks):
            cps[t + 2] = start(t + 2)

        if c == _NCH - 1:
            flat = cidx * _L + lane
            m = jnp.max(cmax)
            cand = jnp.where(cmax == m, flat, jnp.int32(2**31 - 1))
            idx = jnp.min(cand)
            h = idx >> 9
            w = idx & (_W - 1)
            qg = wid * _QPW + q
            cp0 = pltpu.async_copy(off_hbm.at[(qg * 2) * _H + h], row0, semr)
            cp1 = pltpu.async_copy(off_hbm.at[(qg * 2 + 1) * _H + h], row1, semr)
            cp2 = pltpu.async_copy(scl_hbm.at[qg * _H + h], row2, semr)
            cp0.wait()
            cp1.wait()
            cp2.wait()
            wv = jnp.full((_L,), w, jnp.int32)
            o0 = plsc.load_gather(row0, [wv])
            o1 = plsc.load_gather(row1, [wv])
            sv = plsc.load_gather(row2, [wv])
            xs = w.astype(jnp.float32) + o0
            ys = h.astype(jnp.float32) + o1
            sc = jnp.exp(sv * jnp.float32(_LN2))
            base = q * 8
            res = jnp.where(lane == base, xs, res)
            res = jnp.where(lane == base + 1, ys, res)
            res = jnp.where(lane == base + 2, sc, res)
            cmax, cidx = neg_inf, zero_i

    resbuf[...] = res
    pltpu.sync_copy(resbuf, out_hbm.at[wid])


@jax.jit
def _detect(scores2, off2, scl2):
    mesh = plsc.VectorSubcoreMesh(
        core_axis_name="c", subcore_axis_name="s",
        num_cores=_NC, num_subcores=_NS)
    run = functools.partial(
        pl.kernel,
        out_type=jax.ShapeDtypeStruct((_NW, _L), jnp.float32),
        mesh=mesh,
        scratch_types=[
            pltpu.VMEM((_CH,), jnp.float32),
            pltpu.VMEM((_CH,), jnp.float32),
            pltpu.VMEM((_W,), jnp.float32),
            pltpu.VMEM((_W,), jnp.float32),
            pltpu.VMEM((_W,), jnp.float32),
            pltpu.VMEM((_L,), jnp.float32),
            pltpu.SemaphoreType.DMA,
            pltpu.SemaphoreType.DMA,
            pltpu.SemaphoreType.DMA,
        ],
        compiler_params=pltpu.CompilerParams(needs_layout_passes=False),
    )(_sc_body)
    return run(scores2, off2, scl2)


def kernel(scores, scales, offsets, pool_ratio):
    qn = scores.shape[0]
    scores2 = scores.reshape(qn, _HW)
    off2 = offsets.reshape(qn * 2 * _H, _W)
    scl2 = scales.reshape(qn * _H, _W)
    out = _detect(scores2, off2, scl2)
    r = out.reshape(qn, 8)
    pf = jnp.asarray(pool_ratio, jnp.float32)
    positions = (r[:, :2] + 0.5) * pf - 0.5
    sel_scales = r[:, 2]
    return positions, sel_scales


# speedup vs baseline: 2.1876x; 2.1876x over previous
import functools

import jax
import jax.numpy as jnp
from jax import lax
from jax.experimental import pallas as pl
from jax.experimental.pallas import tpu as pltpu
from jax.experimental.pallas import tpu_sc as plsc

_NC = 2
_NS = 16
_L = 16
_NW = _NC * _NS
_QPW = 2
_H = 512
_W = 512
_HW = _H * _W
_CH = 32768
_NCH = _HW // _CH
_VIT = _CH // _L
_UNR = 8
_LN2 = 0.6931471805599453


def _sc_body(scores_hbm, off_hbm, scl_hbm, out_hbm,
             buf0, buf1, row0, row1, row2, resbuf, sem0, sem1, semr):
    wid = lax.axis_index("s") * _NC + lax.axis_index("c")
    lane = lax.iota(jnp.int32, _L)
    bufs = (buf0, buf1)
    sems = (sem0, sem1)

    tasks = [(q, c) for q in range(_QPW) for c in range(_NCH)]

    def start(t):
        q, c = tasks[t]
        src = scores_hbm.at[wid * _QPW + q, pl.ds(c * _CH, _CH)]
        return pltpu.async_copy(src, bufs[t % 2], sems[t % 2])

    cps = [None] * len(tasks)
    cps[0] = start(0)
    cps[1] = start(1)

    neg_inf = jnp.full((_L,), -jnp.inf, jnp.float32)
    zero_i = jnp.zeros((_L,), jnp.int32)
    ms = [neg_inf] * _UNR
    ixs = [zero_i] * _UNR
    res = jnp.zeros((_L,), jnp.float32)

    for t, (q, c) in enumerate(tasks):
        cps[t].wait()
        buf = bufs[t % 2]
        cbase = c * _VIT

        def inner(i, carry, buf=buf, cbase=cbase):
            m, ix = list(carry[0]), list(carry[1])
            for k in range(_UNR):
                j = i * _UNR + k
                v = buf[pl.ds(j * _L, _L)]
                g = v > m[k]
                m[k] = jnp.where(g, v, m[k])
                ix[k] = jnp.where(g, cbase + j, ix[k])
            return tuple(m), tuple(ix)

        acc = lax.fori_loop(0, _VIT // _UNR, inner, (tuple(ms), tuple(ixs)))
        ms, ixs = list(acc[0]), list(acc[1])
        if t + 2 < len(tasks):
            cps[t + 2] = start(t + 2)

        if c == _NCH - 1:
            pairs = [(ms[k], ixs[k] * _L + lane) for k in range(_UNR)]
            while len(pairs) > 1:
                nxt = []
                for a in range(0, len(pairs), 2):
                    (m1, f1), (m2, f2) = pairs[a], pairs[a + 1]
                    take = (m2 > m1) | ((m2 == m1) & (f2 < f1))
                    nxt.append((jnp.where(take, m2, m1),
                                jnp.where(take, f2, f1)))
                pairs = nxt
            cmax, flat = pairs[0]
            m = jnp.max(cmax)
            cand = jnp.where(cmax == m, flat, jnp.int32(2**31 - 1))
            idx = jnp.min(cand)
            h = idx >> 9
            w = idx & (_W - 1)
            qg = wid * _QPW + q
            cp0 = pltpu.async_copy(off_hbm.at[(qg * 2) * _H + h], row0, semr)
            cp1 = pltpu.async_copy(off_hbm.at[(qg * 2 + 1) * _H + h], row1, semr)
            cp2 = pltpu.async_copy(scl_hbm.at[qg * _H + h], row2, semr)
            cp0.wait()
            cp1.wait()
            cp2.wait()
            wv = jnp.full((_L,), w, jnp.int32)
            o0 = plsc.load_gather(row0, [wv])
            o1 = plsc.load_gather(row1, [wv])
            sv = plsc.load_gather(row2, [wv])
            xs = w.astype(jnp.float32) + o0
            ys = h.astype(jnp.float32) + o1
            sc = jnp.exp(sv * jnp.float32(_LN2))
            base = q * 8
            res = jnp.where(lane == base, xs, res)
            res = jnp.where(lane == base + 1, ys, res)
            res = jnp.where(lane == base + 2, sc, res)
            ms = [neg_inf] * _UNR
            ixs = [zero_i] * _UNR

    resbuf[...] = res
    pltpu.sync_copy(resbuf, out_hbm.at[wid])


@jax.jit
def _detect(scores2, off2, scl2):
    mesh = plsc.VectorSubcoreMesh(
        core_axis_name="c", subcore_axis_name="s",
        num_cores=_NC, num_subcores=_NS)
    run = functools.partial(
        pl.kernel,
        out_type=jax.ShapeDtypeStruct((_NW, _L), jnp.float32),
        mesh=mesh,
        scratch_types=[
            pltpu.VMEM((_CH,), jnp.float32),
            pltpu.VMEM((_CH,), jnp.float32),
            pltpu.VMEM((_W,), jnp.float32),
            pltpu.VMEM((_W,), jnp.float32),
            pltpu.VMEM((_W,), jnp.float32),
            pltpu.VMEM((_L,), jnp.float32),
            pltpu.SemaphoreType.DMA,
            pltpu.SemaphoreType.DMA,
            pltpu.SemaphoreType.DMA,
        ],
        compiler_params=pltpu.CompilerParams(needs_layout_passes=False),
    )(_sc_body)
    return run(scores2, off2, scl2)


def kernel(scores, scales, offsets, pool_ratio):
    qn = scores.shape[0]
    scores2 = scores.reshape(qn, _HW)
    off2 = offsets.reshape(qn * 2 * _H, _W)
    scl2 = scales.reshape(qn * _H, _W)
    out = _detect(scores2, off2, scl2)
    r = out.reshape(qn, 8)
    pf = jnp.asarray(pool_ratio, jnp.float32)
    positions = (r[:, :2] + 0.5) * pf - 0.5
    sel_scales = r[:, 2]
    return positions, sel_scales


# speedup vs baseline: 3.2048x; 1.4649x over previous
import functools

import jax
import jax.numpy as jnp
from jax import lax
from jax.experimental import pallas as pl
from jax.experimental.pallas import tpu as pltpu
from jax.experimental.pallas import tpu_sc as plsc

_NC = 2
_NS = 16
_L = 16
_NW = _NC * _NS
_QPW = 2
_H = 512
_W = 512
_HW = _H * _W
_CH = 32768
_NCH = _HW // _CH
_RPC = _CH // _W
_VIT = _CH // _L
_UNR = 8
_LN2 = 0.6931471805599453


def _sc_body(scores_hbm, off_hbm, scl_hbm, out_hbm,
             buf0, buf1, row0, row1, row2, resbuf, sem0, sem1, semr):
    wid = lax.axis_index("s") * _NC + lax.axis_index("c")
    lane = lax.iota(jnp.int32, _L)
    bufs = (buf0, buf1)
    sems = (sem0, sem1)

    tasks = [(q, c) for q in range(_QPW) for c in range(_NCH)]

    def start(t):
        q, c = tasks[t]
        r0 = (wid * _QPW + q) * _H + c * _RPC
        src = scores_hbm.at[pl.ds(r0, _RPC), :]
        return pltpu.async_copy(src, bufs[t % 2], sems[t % 2])

    cps = [None] * len(tasks)
    cps[0] = start(0)
    cps[1] = start(1)

    neg_inf = jnp.full((_L,), -jnp.inf, jnp.float32)
    zero_i = jnp.zeros((_L,), jnp.int32)
    ms = [neg_inf] * _UNR
    ixs = [zero_i] * _UNR
    res = jnp.zeros((_L,), jnp.float32)

    for t, (q, c) in enumerate(tasks):
        cps[t].wait()
        buf = bufs[t % 2]
        cbase = c * _VIT

        def inner(r, carry, buf=buf, cbase=cbase):
            m, ix = list(carry[0]), list(carry[1])
            rbase = cbase + r * (_W // _L)
            for k in range(_W // _L):
                v = buf[r, pl.ds(k * _L, _L)]
                a = k % _UNR
                g = v > m[a]
                m[a] = jnp.where(g, v, m[a])
                ix[a] = jnp.where(g, rbase + k, ix[a])
            return tuple(m), tuple(ix)

        acc = lax.fori_loop(0, _RPC, inner, (tuple(ms), tuple(ixs)))
        ms, ixs = list(acc[0]), list(acc[1])
        if t + 2 < len(tasks):
            cps[t + 2] = start(t + 2)

        if c == _NCH - 1:
            pairs = [(ms[k], ixs[k] * _L + lane) for k in range(_UNR)]
            while len(pairs) > 1:
                nxt = []
                for a in range(0, len(pairs), 2):
                    (m1, f1), (m2, f2) = pairs[a], pairs[a + 1]
                    take = (m2 > m1) | ((m2 == m1) & (f2 < f1))
                    nxt.append((jnp.where(take, m2, m1),
                                jnp.where(take, f2, f1)))
                pairs = nxt
            cmax, flat = pairs[0]
            m = jnp.max(cmax)
            cand = jnp.where(cmax == m, flat, jnp.int32(2**31 - 1))
            idx = jnp.min(cand)
            h = idx >> 9
            w = idx & (_W - 1)
            qg = wid * _QPW + q
            cp0 = pltpu.async_copy(off_hbm.at[(qg * 2) * _H + h], row0, semr)
            cp1 = pltpu.async_copy(off_hbm.at[(qg * 2 + 1) * _H + h], row1, semr)
            cp2 = pltpu.async_copy(scl_hbm.at[qg * _H + h], row2, semr)
            cp0.wait()
            cp1.wait()
            cp2.wait()
            wv = jnp.full((_L,), w, jnp.int32)
            o0 = plsc.load_gather(row0, [wv])
            o1 = plsc.load_gather(row1, [wv])
            sv = plsc.load_gather(row2, [wv])
            xs = w.astype(jnp.float32) + o0
            ys = h.astype(jnp.float32) + o1
            sc = jnp.exp(sv * jnp.float32(_LN2))
            base = q * 8
            res = jnp.where(lane == base, xs, res)
            res = jnp.where(lane == base + 1, ys, res)
            res = jnp.where(lane == base + 2, sc, res)
            ms = [neg_inf] * _UNR
            ixs = [zero_i] * _UNR

    resbuf[...] = res
    pltpu.sync_copy(resbuf, out_hbm.at[wid])


@jax.jit
def _detect(scores2, off2, scl2):
    mesh = plsc.VectorSubcoreMesh(
        core_axis_name="c", subcore_axis_name="s",
        num_cores=_NC, num_subcores=_NS)
    run = functools.partial(
        pl.kernel,
        out_type=jax.ShapeDtypeStruct((_NW, _L), jnp.float32),
        mesh=mesh,
        scratch_types=[
            pltpu.VMEM((_RPC, _W), jnp.float32),
            pltpu.VMEM((_RPC, _W), jnp.float32),
            pltpu.VMEM((_W,), jnp.float32),
            pltpu.VMEM((_W,), jnp.float32),
            pltpu.VMEM((_W,), jnp.float32),
            pltpu.VMEM((_L,), jnp.float32),
            pltpu.SemaphoreType.DMA,
            pltpu.SemaphoreType.DMA,
            pltpu.SemaphoreType.DMA,
        ],
        compiler_params=pltpu.CompilerParams(needs_layout_passes=False),
    )(_sc_body)
    return run(scores2, off2, scl2)


def kernel(scores, scales, offsets, pool_ratio):
    qn = scores.shape[0]
    scores2 = scores.reshape(qn * _H, _W)
    off2 = offsets.reshape(qn * 2 * _H, _W)
    scl2 = scales.reshape(qn * _H, _W)
    out = _detect(scores2, off2, scl2)
    r = out.reshape(qn, 8)
    pf = jnp.asarray(pool_ratio, jnp.float32)
    positions = (r[:, :2] + 0.5) * pf - 0.5
    sel_scales = r[:, 2]
    return positions, sel_scales


# speedup vs baseline: 4.2476x; 1.3254x over previous
import functools

import jax
import jax.numpy as jnp
from jax import lax
from jax.experimental import pallas as pl
from jax.experimental.pallas import tpu as pltpu
from jax.experimental.pallas import tpu_sc as plsc

_NC = 2
_NS = 16
_L = 16
_NW = _NC * _NS
_QPW = 2
_H = 512
_W = 512
_HW = _H * _W
_CH = 32768
_NCH = _HW // _CH
_RPC = _CH // _W
_VIT = _CH // _L
_UNR = 8
_LN2 = 0.6931471805599453


def _sc_body(scores_hbm, off_hbm, scl_hbm, out_hbm,
             buf0, buf1, row0, row1, row2, resbuf, sem0, sem1, semr):
    wid = lax.axis_index("s") * _NC + lax.axis_index("c")
    lane = lax.iota(jnp.int32, _L)
    bufs = (buf0, buf1)
    sems = (sem0, sem1)

    tasks = [(q, c) for q in range(_QPW) for c in range(_NCH)]

    def start(t):
        q, c = tasks[t]
        r0 = (wid * _QPW + q) * _H + c * _RPC
        src = scores_hbm.at[pl.ds(r0, _RPC), :]
        return pltpu.async_copy(src, bufs[t % 2], sems[t % 2])

    cps = [None] * len(tasks)
    cps[0] = start(0)
    cps[1] = start(1)

    neg_inf = jnp.full((_L,), -jnp.inf, jnp.float32)
    zero_i = jnp.zeros((_L,), jnp.int32)
    ms = [neg_inf] * _UNR
    ixs = [zero_i] * _UNR
    res = jnp.zeros((_L,), jnp.float32)

    for t, (q, c) in enumerate(tasks):
        cps[t].wait()
        buf = bufs[t % 2]
        cbase = c * _VIT

        def inner(i, carry, buf=buf, cbase=cbase):
            m, ix = list(carry[0]), list(carry[1])
            r = i >> 2
            cw = (i & 3) * (_UNR * _L)
            for k in range(_UNR):
                v = buf[r, pl.ds(cw + k * _L, _L)]
                g = v > m[k]
                m[k] = jnp.where(g, v, m[k])
                ix[k] = jnp.where(g, cbase + i * _UNR + k, ix[k])
            return tuple(m), tuple(ix)

        acc = lax.fori_loop(0, _VIT // _UNR, inner, (tuple(ms), tuple(ixs)))
        ms, ixs = list(acc[0]), list(acc[1])
        if t + 2 < len(tasks):
            cps[t + 2] = start(t + 2)

        if c == _NCH - 1:
            pairs = [(ms[k], ixs[k] * _L + lane) for k in range(_UNR)]
            while len(pairs) > 1:
                nxt = []
                for a in range(0, len(pairs), 2):
                    (m1, f1), (m2, f2) = pairs[a], pairs[a + 1]
                    take = (m2 > m1) | ((m2 == m1) & (f2 < f1))
                    nxt.append((jnp.where(take, m2, m1),
                                jnp.where(take, f2, f1)))
                pairs = nxt
            cmax, flat = pairs[0]
            m = jnp.max(cmax)
            cand = jnp.where(cmax == m, flat, jnp.int32(2**31 - 1))
            idx = jnp.min(cand)
            h = idx >> 9
            w = idx & (_W - 1)
            qg = wid * _QPW + q
            cp0 = pltpu.async_copy(off_hbm.at[(qg * 2) * _H + h], row0, semr)
            cp1 = pltpu.async_copy(off_hbm.at[(qg * 2 + 1) * _H + h], row1, semr)
            cp2 = pltpu.async_copy(scl_hbm.at[qg * _H + h], row2, semr)
            cp0.wait()
            cp1.wait()
            cp2.wait()
            wv = jnp.full((_L,), w, jnp.int32)
            o0 = plsc.load_gather(row0, [wv])
            o1 = plsc.load_gather(row1, [wv])
            sv = plsc.load_gather(row2, [wv])
            xs = w.astype(jnp.float32) + o0
            ys = h.astype(jnp.float32) + o1
            sc = jnp.exp(sv * jnp.float32(_LN2))
            base = q * 8
            res = jnp.where(lane == base, xs, res)
            res = jnp.where(lane == base + 1, ys, res)
            res = jnp.where(lane == base + 2, sc, res)
            ms = [neg_inf] * _UNR
            ixs = [zero_i] * _UNR

    resbuf[...] = res
    pltpu.sync_copy(resbuf, out_hbm.at[wid])


@jax.jit
def _detect(scores2, off2, scl2):
    mesh = plsc.VectorSubcoreMesh(
        core_axis_name="c", subcore_axis_name="s",
        num_cores=_NC, num_subcores=_NS)
    run = functools.partial(
        pl.kernel,
        out_type=jax.ShapeDtypeStruct((_NW, _L), jnp.float32),
        mesh=mesh,
        scratch_types=[
            pltpu.VMEM((_RPC, _W), jnp.float32),
            pltpu.VMEM((_RPC, _W), jnp.float32),
            pltpu.VMEM((_W,), jnp.float32),
            pltpu.VMEM((_W,), jnp.float32),
            pltpu.VMEM((_W,), jnp.float32),
            pltpu.VMEM((_L,), jnp.float32),
            pltpu.SemaphoreType.DMA,
            pltpu.SemaphoreType.DMA,
            pltpu.SemaphoreType.DMA,
        ],
        compiler_params=pltpu.CompilerParams(needs_layout_passes=False),
    )(_sc_body)
    return run(scores2, off2, scl2)


def kernel(scores, scales, offsets, pool_ratio):
    qn = scores.shape[0]
    scores2 = scores.reshape(qn * _H, _W)
    off2 = offsets.reshape(qn * 2 * _H, _W)
    scl2 = scales.reshape(qn * _H, _W)
    out = _detect(scores2, off2, scl2)
    r = out.reshape(qn, 8)
    pf = jnp.asarray(pool_ratio, jnp.float32)
    positions = (r[:, :2] + 0.5) * pf - 0.5
    sel_scales = r[:, 2]
    return positions, sel_scales


# speedup vs baseline: 4.2477x; 1.0000x over previous
import functools

import jax
import jax.numpy as jnp
from jax import lax
from jax.experimental import pallas as pl
from jax.experimental.pallas import tpu as pltpu
from jax.experimental.pallas import tpu_sc as plsc

_NC = 2
_NS = 16
_L = 16
_NW = _NC * _NS
_QPW = 2
_H = 512
_W = 512
_HW = _H * _W
_CH = 32768
_NCH = _HW // _CH
_RPC = _CH // _W
_VIT = _CH // _L
_UNR = 8
_LN2 = 0.6931471805599453


def _sc_body(scores_hbm, off_hbm, scl_hbm, out_hbm,
             buf0, buf1, row0, row1, row2, resbuf, sem0, sem1, semr):
    wid = lax.axis_index("s") * _NC + lax.axis_index("c")
    lane = lax.iota(jnp.int32, _L)
    bufs = (buf0, buf1)
    sems = (sem0, sem1)

    tasks = [(q, c) for q in range(_QPW) for c in range(_NCH)]

    def start(t):
        q, c = tasks[t]
        r0 = (wid * _QPW + q) * _H + c * _RPC
        src = scores_hbm.at[pl.ds(r0, _RPC), :]
        return pltpu.async_copy(src, bufs[t % 2], sems[t % 2])

    cps = [None] * len(tasks)
    cps[0] = start(0)
    cps[1] = start(1)

    neg_inf = jnp.full((_L,), -jnp.inf, jnp.float32)
    zero_i = jnp.zeros((_L,), jnp.int32)
    ms = [neg_inf] * _UNR
    ixs = [zero_i] * _UNR
    res = jnp.zeros((_L,), jnp.float32)

    for t, (q, c) in enumerate(tasks):
        cps[t].wait()
        buf = bufs[t % 2]
        cbase = c * _VIT

        def inner(i, carry, buf=buf, cbase=cbase):
            m, ix = list(carry[0]), list(carry[1])
            r = i >> 2
            cw = (i & 3) * (_UNR * _L)
            for k in range(_UNR):
                v = buf[r, pl.ds(cw + k * _L, _L)]
                g = v > m[k]
                m[k] = jnp.where(g, v, m[k])
                ix[k] = jnp.where(g, cbase + i * _UNR + k, ix[k])
            return tuple(m), tuple(ix)

        acc = lax.fori_loop(0, _VIT // _UNR, inner, (tuple(ms), tuple(ixs)))
        ms, ixs = list(acc[0]), list(acc[1])
        if t + 2 < len(tasks):
            cps[t + 2] = start(t + 2)

        if c == _NCH - 1:
            pairs = [(ms[k], ixs[k] * _L + lane) for k in range(_UNR)]
            while len(pairs) > 1:
                nxt = []
                for a in range(0, len(pairs), 2):
                    (m1, f1), (m2, f2) = pairs[a], pairs[a + 1]
                    take = (m2 > m1) | ((m2 == m1) & (f2 < f1))
                    nxt.append((jnp.where(take, m2, m1),
                                jnp.where(take, f2, f1)))
                pairs = nxt
            cmax, flat = pairs[0]
            m = jnp.max(cmax)
            cand = jnp.where(cmax == m, flat, jnp.int32(2**31 - 1))
            idx = jnp.min(cand)
            h = idx >> 9
            w = idx & (_W - 1)
            qg = wid * _QPW + q
            cp0 = pltpu.async_copy(off_hbm.at[(qg * 2) * _H + h], row0, semr)
            cp1 = pltpu.async_copy(off_hbm.at[(qg * 2 + 1) * _H + h], row1, semr)
            cp2 = pltpu.async_copy(scl_hbm.at[qg * _H + h], row2, semr)
            cp0.wait()
            cp1.wait()
            cp2.wait()
            wv = jnp.full((_L,), w, jnp.int32)
            o0 = plsc.load_gather(row0, [wv])
            o1 = plsc.load_gather(row1, [wv])
            sv = plsc.load_gather(row2, [wv])
            xs = w.astype(jnp.float32) + o0
            ys = h.astype(jnp.float32) + o1
            sc = jnp.exp(sv * jnp.float32(_LN2))
            base = q * 8
            res = jnp.where(lane == base, xs, res)
            res = jnp.where(lane == base + 1, ys, res)
            res = jnp.where(lane == base + 2, sc, res)
            ms = [neg_inf] * _UNR
            ixs = [zero_i] * _UNR

    resbuf[...] = res
    pltpu.sync_copy(resbuf, out_hbm.at[pl.ds(wid * _L, _L)])


@jax.jit
def _detect(scores2, off2, scl2):
    mesh = plsc.VectorSubcoreMesh(
        core_axis_name="c", subcore_axis_name="s",
        num_cores=_NC, num_subcores=_NS)
    run = functools.partial(
        pl.kernel,
        out_type=jax.ShapeDtypeStruct((_NW * _L,), jnp.float32),
        mesh=mesh,
        scratch_types=[
            pltpu.VMEM((_RPC, _W), jnp.float32),
            pltpu.VMEM((_RPC, _W), jnp.float32),
            pltpu.VMEM((_W,), jnp.float32),
            pltpu.VMEM((_W,), jnp.float32),
            pltpu.VMEM((_W,), jnp.float32),
            pltpu.VMEM((_L,), jnp.float32),
            pltpu.SemaphoreType.DMA,
            pltpu.SemaphoreType.DMA,
            pltpu.SemaphoreType.DMA,
        ],
        compiler_params=pltpu.CompilerParams(needs_layout_passes=False),
    )(_sc_body)
    return run(scores2, off2, scl2)


def kernel(scores, scales, offsets, pool_ratio):
    qn = scores.shape[0]
    scores2 = scores.reshape(qn * _H, _W)
    off2 = offsets.reshape(qn * 2 * _H, _W)
    scl2 = scales.reshape(qn * _H, _W)
    out = _detect(scores2, off2, scl2)
    r = out.reshape(qn, 8)
    pf = jnp.asarray(pool_ratio, jnp.float32)
    positions = (r[:, :2] + 0.5) * pf - 0.5
    sel_scales = r[:, 2]
    return positions, sel_scales
